# baseline (speedup 1.0000x reference)
import functools
import jax
import jax.numpy as jnp
import numpy as np
from jax import lax
from jax.experimental import pallas as pl
from jax.experimental.pallas import tpu as pltpu
from jax.experimental.pallas import tpu_sc as plsc

N = 10000
E = 160000
G = 64
NP = 10016
EP = 163840
NW = 32
RPW = EP // NW
CH = 128
NCH = RPW // CH
ET = 512
EG = EP // ET
W = 128

F32 = jnp.float32
_SDS = jax.ShapeDtypeStruct


def _full(shape):
    return pl.BlockSpec(shape, lambda i: (0,) * len(shape))


def _edge_mlp_body(ea, w1a, b1a, w2a, b2a, w1b, b1b, w2b, b2b, oa, ob):
    e = ea[...]
    ha = jnp.maximum(jnp.dot(e, w1a[...], preferred_element_type=F32) + b1a[...], 0.0)
    oa[...] = jnp.maximum(jnp.dot(ha, w2a[...], preferred_element_type=F32) + b2a[...], 0.0)
    hb = jnp.maximum(jnp.dot(e, w1b[...], preferred_element_type=F32) + b1b[...], 0.0)
    ob[...] = jnp.maximum(jnp.dot(hb, w2b[...], preferred_element_type=F32) + b2b[...], 0.0)


def _edge_mlp(eaP, w1a, b1a, w2a, b2a, w1b, b1b, w2b, b2b):
    espec = pl.BlockSpec((ET, 16), lambda i: (i, 0))
    ospec = pl.BlockSpec((ET, 64), lambda i: (i, 0))
    return pl.pallas_call(
        _edge_mlp_body,
        grid=(EG,),
        in_specs=[espec, _full((16, 128)), _full((1, 128)), _full((128, 64)),
                  _full((1, 64)), _full((16, 128)), _full((1, 128)),
                  _full((128, 64)), _full((1, 64))],
        out_specs=[ospec, ospec],
        out_shape=[_SDS((EP, 64), F32), _SDS((EP, 64), F32)],
    )(eaP, w1a, b1a, w2a, b2a, w1b, b1b, w2b, b2b)


def _conv_edge_body(cin, nrep, xs, h2, w3p, smat, b3r, msg):
    xv = xs[...][:, :cin]
    p = jnp.dot(xv, w3p[...], preferred_element_type=F32)
    h2t = jnp.concatenate([h2[...]] * nrep, axis=1)
    m16 = (jnp.dot(p * h2t, smat[...], preferred_element_type=F32)
           + jnp.dot(xv, b3r[...], preferred_element_type=F32))
    msg[...] = jnp.concatenate([m16, jnp.zeros((ET, W - 16), F32)], axis=1)


def _conv_edge(xs, h2, w3p, smat, b3r, cin, nrep):
    ow = 64 * nrep
    return pl.pallas_call(
        functools.partial(_conv_edge_body, cin, nrep),
        grid=(EG,),
        in_specs=[pl.BlockSpec((ET, W), lambda i: (i, 0)),
                  pl.BlockSpec((ET, 64), lambda i: (i, 0)),
                  _full((cin, ow)), _full((ow, 16)), _full((cin, 16))],
        out_specs=pl.BlockSpec((ET, W), lambda i: (i, 0)),
        out_shape=_SDS((EP, W), F32),
    )(xs, h2, w3p, smat, b3r)


def _node1_body(part, x, rootp, biasp, lin2, atts, attd, hgp, as_o, ad_o, m_o):
    agg = part[0, :, :16] + part[1, :, :16]
    h1 = jnp.maximum(agg + jnp.dot(x[...], rootp[...], preferred_element_type=F32)
                     + biasp[...], 0.0)
    hg = jnp.dot(h1, lin2[...], preferred_element_type=F32)
    hgp[...] = jnp.concatenate(
        [hg, jnp.ones((NP, 1), F32), jnp.zeros((NP, W - 17), F32)], axis=1)
    a_s = jnp.sum(hg * atts[...], axis=1, keepdims=True)
    a_d = jnp.sum(hg * attd[...], axis=1, keepdims=True)
    as_o[...] = a_s
    ad_o[...] = a_d
    m = jnp.max(a_s) + jnp.max(a_d)
    m = jnp.where(m > 0, m, 0.2 * m)
    m_o[...] = jnp.broadcast_to(m, (1, 16))


def _node1(part, xP, rootp, biasp, lin2, atts, attd):
    return pl.pallas_call(
        _node1_body,
        grid=(1,),
        in_specs=[_full((2, NP, W)), _full((NP, 128)), _full((128, 16)),
                  _full((1, 16)), _full((16, 16)), _full((1, 16)), _full((1, 16))],
        out_specs=[_full((NP, W)), _full((NP, 1)), _full((NP, 1)), _full((1, 16))],
        out_shape=[_SDS((NP, W), F32), _SDS((NP, 1), F32), _SDS((NP, 1), F32),
                   _SDS((1, 16), F32)],
    )(part, xP, rootp, biasp, lin2, atts, attd)


def _node2_body(out2, bias2, root3, bias3, h2n_o, xr3b_o):
    osum = out2[0] + out2[1]
    dinv = 1.0 / (osum[:, 16:17] + 1e-16)
    h2n = jnp.maximum(osum[:, :16] * dinv + bias2[...], 0.0)
    h2n_o[...] = jnp.concatenate([h2n, jnp.zeros((NP, W - 16), F32)], axis=1)
    xr3b_o[...] = jnp.dot(h2n, root3[...], preferred_element_type=F32) + bias3[...]


def _node2(out2, bias2, root3, bias3):
    return pl.pallas_call(
        _node2_body,
        grid=(1,),
        in_specs=[_full((2, NP, W)), _full((1, 16)),
                  _full((16, 16)), _full((1, 16))],
        out_specs=[_full((NP, W)), _full((NP, 16))],
        out_shape=[_SDS((NP, W), F32), _SDS((NP, 16), F32)],
    )(out2, bias2, root3, bias3)


def _node3_body(part, xr3b, lin4, atts, attd, hgp, as_o, ad_o, m_o):
    h3 = jnp.maximum(part[0, :, :16] + part[1, :, :16] + xr3b[...], 0.0)
    hg = jnp.dot(h3, lin4[...], preferred_element_type=F32)
    hgp[...] = jnp.concatenate(
        [hg, jnp.ones((NP, 1), F32), jnp.zeros((NP, W - 33), F32)], axis=1)
    a_s = jnp.sum(hg * atts[...], axis=1, keepdims=True)
    a_d = jnp.sum(hg * attd[...], axis=1, keepdims=True)
    as_o[...] = a_s
    ad_o[...] = a_d
    m = jnp.max(a_s) + jnp.max(a_d)
    m = jnp.where(m > 0, m, 0.2 * m)
    m_o[...] = jnp.broadcast_to(m, (1, 16))


def _node3(part, xr3b, lin4, atts, attd):
    return pl.pallas_call(
        _node3_body,
        grid=(1,),
        in_specs=[_full((2, NP, W)), _full((NP, 16)), _full((16, 32)),
                  _full((1, 32)), _full((1, 32))],
        out_specs=[_full((NP, W)), _full((NP, 1)), _full((NP, 1)), _full((1, 16))],
        out_shape=[_SDS((NP, W), F32), _SDS((NP, 1), F32), _SDS((NP, 1), F32),
                   _SDS((1, 16), F32)],
    )(part, xr3b, lin4, atts, attd)


def _head_body(out4, bias4, batch, fc1, fc1b, fc2, fc2b, res):
    osum = out4[0] + out4[1]
    dinv = 1.0 / (osum[:, 32:33] + 1e-16)
    h4 = jnp.maximum(osum[:, :32] * dinv + bias4[...], 0.0)
    gids = lax.broadcasted_iota(jnp.int32, (1, G), 1)
    oh = (batch[...] == gids).astype(F32)
    counts = jnp.sum(oh, axis=0, keepdims=True)
    pooled = lax.dot_general(oh, h4, (((0,), (0,)), ((), ())),
                             preferred_element_type=F32,
                             precision=lax.Precision.HIGHEST)
    pooled = pooled / jnp.maximum(counts, 1.0).T
    z = jnp.maximum(jnp.dot(pooled, fc1[...], preferred_element_type=F32)
                    + fc1b[...], 0.0)
    res[...] = jnp.dot(z, fc2[...], preferred_element_type=F32) + fc2b[...]


def _head(out4, bias4, batchP, fc1, fc1b, fc2, fc2b):
    return pl.pallas_call(
        _head_body,
        grid=(1,),
        in_specs=[_full((2, NP, W)), _full((1, 32)),
                  _full((NP, 1)), _full((32, 64)), _full((1, 64)),
                  _full((64, 1)), _full((1, 1))],
        out_specs=_full((G, 1)),
        out_shape=_SDS((G, 1), F32),
    )(out4, bias4, batchP, fc1, fc1b, fc2, fc2b)


@functools.lru_cache(maxsize=None)
def _mesh():
    return plsc.VectorSubcoreMesh(core_axis_name="c", subcore_axis_name="s",
                                  num_cores=2, num_subcores=16)


def _wid():
    return lax.axis_index("s") * 2 + lax.axis_index("c")


@functools.lru_cache(maxsize=None)
def _sc_gather():
    def body(table_hbm, idx_hbm, out_hbm, idxv, buf, tsp, sem):
        wid = _wid()
        pltpu.sync_copy(idx_hbm.at[wid], idxv)

        @pl.when(lax.axis_index("s") == 0)
        def _():
            pltpu.sync_copy(table_hbm, tsp)

        plsc.subcore_barrier()

        @pl.loop(0, NCH)
        def _(i):
            pltpu.async_copy(tsp.at[idxv.at[i]], buf, sem).wait()
            pltpu.sync_copy(buf, out_hbm.at[pl.ds(wid * RPW + i * CH, CH)])

    return pl.kernel(
        body,
        out_type=_SDS((EP, W), F32),
        mesh=_mesh(),
        compiler_params=pltpu.CompilerParams(needs_layout_passes=False),
        scratch_types=[
            pltpu.VMEM((NCH, CH), jnp.int32),
            pltpu.VMEM((CH, W), F32),
            pltpu.VMEM_SHARED((NP, W), F32),
            pltpu.SemaphoreType.DMA,
        ],
    )


@functools.lru_cache(maxsize=None)
def _sc_scatter_add():
    def body(vals_hbm, idx_hbm, zeros_hbm, out_hbm, idxv, buf, accsp):
        wid = _wid()
        pltpu.sync_copy(idx_hbm.at[wid], idxv)

        @pl.when(lax.axis_index("s") == 0)
        def _():
            pltpu.sync_copy(zeros_hbm, accsp)

        plsc.subcore_barrier()

        @pl.loop(0, NCH)
        def _(i):
            pltpu.sync_copy(vals_hbm.at[pl.ds(wid * RPW + i * CH, CH)], buf)
            pltpu.sync_copy(buf, accsp.at[idxv.at[i]], add=True)

        plsc.subcore_barrier()

        @pl.when(lax.axis_index("s") == 0)
        def _():
            pltpu.sync_copy(accsp, out_hbm.at[lax.axis_index("c")])

    return pl.kernel(
        body,
        out_type=_SDS((2, NP, W), F32),
        mesh=_mesh(),
        compiler_params=pltpu.CompilerParams(needs_layout_passes=False),
        scratch_types=[
            pltpu.VMEM((NCH, CH), jnp.int32),
            pltpu.VMEM((CH, W), F32),
            pltpu.VMEM_SHARED((NP, W), F32),
        ],
    )


@functools.lru_cache(maxsize=None)
def _sc_gat(nscale):
    def body(as_hbm, ad_hbm, hgp_hbm, m_hbm, sidx_hbm, didx_hbm, zw_hbm,
             out_hbm, asv, adv, mv, sidxv, didxv, exbuf, hrows, osp, sem):
        wid = _wid()
        pltpu.sync_copy(as_hbm, asv)
        pltpu.sync_copy(ad_hbm, adv)
        pltpu.sync_copy(m_hbm, mv)
        pltpu.sync_copy(sidx_hbm.at[wid], sidxv)
        pltpu.sync_copy(didx_hbm.at[wid], didxv)

        @pl.when(lax.axis_index("s") == 0)
        def _():
            pltpu.sync_copy(zw_hbm, osp)

        plsc.subcore_barrier()
        mvec = mv[0, :]

        @pl.loop(0, NCH)
        def _(i):
            srow = sidxv.at[i]
            drow = didxv.at[i]
            pltpu.async_copy(hgp_hbm.at[srow], hrows, sem).wait()
            for j in range(CH // 16):
                sidx = srow[pl.ds(16 * j, 16)]
                didx = drow[pl.ds(16 * j, 16)]
                z = plsc.load_gather(asv, [sidx]) + plsc.load_gather(adv, [didx])
                a = jnp.where(z > 0, z, 0.2 * z)
                exbuf[pl.ds(16 * j, 16)] = jnp.exp(a - mvec)

            @pl.loop(0, CH)
            def _(r):
                cf = plsc.load_gather(exbuf, [lax.broadcast(r, (16,))])
                rrow = hrows.at[r]
                for h in range(nscale):
                    rrow[pl.ds(16 * h, 16)] = rrow[pl.ds(16 * h, 16)] * cf

            pltpu.sync_copy(hrows, osp.at[drow], add=True)

        plsc.subcore_barrier()

        @pl.when(lax.axis_index("s") == 0)
        def _():
            pltpu.sync_copy(osp, out_hbm.at[lax.axis_index("c")])

    return pl.kernel(
        body,
        out_type=_SDS((2, NP, W), F32),
        mesh=_mesh(),
        compiler_params=pltpu.CompilerParams(needs_layout_passes=False),
        scratch_types=[
            pltpu.VMEM((NP,), F32),
            pltpu.VMEM((NP,), F32),
            pltpu.VMEM((1, 16), F32),
            pltpu.VMEM((NCH, CH), jnp.int32),
            pltpu.VMEM((NCH, CH), jnp.int32),
            pltpu.VMEM((CH,), F32),
            pltpu.VMEM((CH, W), F32),
            pltpu.VMEM_SHARED((NP, W), F32),
            pltpu.SemaphoreType.DMA,
        ],
    )


def _gather_rows(*a):
    return _sc_gather()(*a)


def _scatter_rows(*a):
    return _sc_scatter_add()(*a)


def _gat2pass(*a):
    return _sc_gat(2)(*a)


def _gat3pass(*a):
    return _sc_gat(3)(*a)


def _w3_omajor(w3, cin, cout):
    return jnp.reshape(jnp.transpose(jnp.reshape(w3, (64, cin, cout)),
                                     (1, 2, 0)), (cin, cout * 64))


def _smat(cout):
    s = np.kron(np.eye(cout, dtype=np.float32), np.ones((64, 1), np.float32))
    return jnp.asarray(np.pad(s, ((0, 0), (0, 16 - cout))))


def _b3r(b3, cin, cout):
    return jnp.pad(jnp.reshape(b3, (cin, cout)), ((0, 0), (0, 16 - cout)))


def kernel(x, edge_index, edge_attr, batch, emlp1_w1, emlp1_b1, emlp1_w2,
           emlp1_b2, emlp1_w3, emlp1_b3, conv1_root, conv1_bias, gat2_lin,
           gat2_att_src, gat2_att_dst, gat2_bias, emlp2_w1, emlp2_b1,
           emlp2_w2, emlp2_b2, emlp2_w3, emlp2_b3, conv3_root, conv3_bias,
           gat4_lin, gat4_att_src, gat4_att_dst, gat4_bias, fc1_w, fc1_b,
           fc2_w, fc2_b):
    xP = jnp.pad(x, ((0, NP - N), (0, 0)))
    eaP = jnp.pad(edge_attr, ((0, EP - E), (0, 0)))
    src = jnp.pad(edge_index[0], (0, EP - E), constant_values=N)
    dst = jnp.pad(edge_index[1], (0, EP - E), constant_values=N)
    sidx3 = jnp.reshape(src, (NW, NCH, CH))
    didx3 = jnp.reshape(dst, (NW, NCH, CH))
    batchP = jnp.pad(batch, (0, NP - N), constant_values=G)[:, None]

    zw = jnp.zeros((NP, W), F32)

    w3p1 = _w3_omajor(emlp1_w3, 128, 8)
    s1 = _smat(8)
    b3r1 = _b3r(emlp1_b3, 128, 8)
    w3p2 = _w3_omajor(emlp2_w3, 16, 16)
    s2 = _smat(16)
    b3r2 = _b3r(emlp2_b3, 16, 16)
    root1p = jnp.pad(conv1_root, ((0, 0), (0, 8)))
    bias1p = jnp.pad(conv1_bias, (0, 8))[None, :]
    lin2p = jnp.pad(gat2_lin, ((0, 8), (0, 0)))

    h2a, h2b = _edge_mlp(eaP, emlp1_w1, emlp1_b1[None, :], emlp1_w2,
                         emlp1_b2[None, :], emlp2_w1, emlp2_b1[None, :],
                         emlp2_w2, emlp2_b2[None, :])
    xs1 = _gather_rows(xP, sidx3)

    msg1 = _conv_edge(xs1, h2a, w3p1, s1, b3r1, 128, 8)
    part1 = _scatter_rows(msg1, didx3, zw)
    hgp2, as2, ad2, m2 = _node1(part1, xP, root1p, bias1p, lin2p,
                                gat2_att_src[None, :], gat2_att_dst[None, :])

    out2 = _gat2pass(jnp.reshape(as2, (NP,)), jnp.reshape(ad2, (NP,)),
                     hgp2, m2, sidx3, didx3, zw)
    h2n, xr3b = _node2(out2, gat2_bias[None, :], conv3_root,
                       conv3_bias[None, :])

    xs3 = _gather_rows(h2n, sidx3)
    msg3 = _conv_edge(xs3, h2b, w3p2, s2, b3r2, 16, 16)
    part3 = _scatter_rows(msg3, didx3, zw)
    hgp4, as4, ad4, m4 = _node3(part3, xr3b, gat4_lin,
                                gat4_att_src[None, :], gat4_att_dst[None, :])

    out4 = _gat3pass(jnp.reshape(as4, (NP,)), jnp.reshape(ad4, (NP,)),
                     hgp4, m4, sidx3, didx3, zw)
    res = _head(out4, gat4_bias[None, :], batchP, fc1_w, fc1_b[None, :],
                fc2_w, jnp.reshape(fc2_b, (1, 1)))
    return jnp.reshape(res, (G,))


# speedup vs baseline: 1.0182x; 1.0182x over previous
import functools
import jax
import jax.numpy as jnp
import numpy as np
from jax import lax
from jax.experimental import pallas as pl
from jax.experimental.pallas import tpu as pltpu
from jax.experimental.pallas import tpu_sc as plsc

N = 10000
E = 160000
G = 64
NP = 10016
EP = 163840
NW = 32
RPW = EP // NW
CH = 128
NCH = RPW // CH
ET = 512
EG = EP // ET
W = 128

F32 = jnp.float32
_SDS = jax.ShapeDtypeStruct


def _full(shape):
    return pl.BlockSpec(shape, lambda i: (0,) * len(shape))


def _edge_mlp_body(ea, w1a, b1a, w2a, b2a, w1b, b1b, w2b, b2b, oa, ob):
    e = ea[...]
    ha = jnp.maximum(jnp.dot(e, w1a[...], preferred_element_type=F32) + b1a[...], 0.0)
    oa[...] = jnp.maximum(jnp.dot(ha, w2a[...], preferred_element_type=F32) + b2a[...], 0.0)
    hb = jnp.maximum(jnp.dot(e, w1b[...], preferred_element_type=F32) + b1b[...], 0.0)
    ob[...] = jnp.maximum(jnp.dot(hb, w2b[...], preferred_element_type=F32) + b2b[...], 0.0)


def _edge_mlp(eaP, w1a, b1a, w2a, b2a, w1b, b1b, w2b, b2b):
    espec = pl.BlockSpec((ET, 16), lambda i: (i, 0))
    ospec = pl.BlockSpec((ET, 64), lambda i: (i, 0))
    return pl.pallas_call(
        _edge_mlp_body,
        grid=(EG,),
        in_specs=[espec, _full((16, 128)), _full((1, 128)), _full((128, 64)),
                  _full((1, 64)), _full((16, 128)), _full((1, 128)),
                  _full((128, 64)), _full((1, 64))],
        out_specs=[ospec, ospec],
        out_shape=[_SDS((EP, 64), F32), _SDS((EP, 64), F32)],
    )(eaP, w1a, b1a, w2a, b2a, w1b, b1b, w2b, b2b)


def _conv_edge_body(cin, nrep, xs, h2, w3p, smat, b3r, msg):
    xv = xs[...][:, :cin]
    p = jnp.dot(xv, w3p[...], preferred_element_type=F32)
    h2t = jnp.concatenate([h2[...]] * nrep, axis=1)
    m16 = (jnp.dot(p * h2t, smat[...], preferred_element_type=F32)
           + jnp.dot(xv, b3r[...], preferred_element_type=F32))
    msg[...] = jnp.concatenate([m16, jnp.zeros((ET, W - 16), F32)], axis=1)


def _conv_edge(xs, h2, w3p, smat, b3r, cin, nrep):
    ow = 64 * nrep
    return pl.pallas_call(
        functools.partial(_conv_edge_body, cin, nrep),
        grid=(EG,),
        in_specs=[pl.BlockSpec((ET, W), lambda i: (i, 0)),
                  pl.BlockSpec((ET, 64), lambda i: (i, 0)),
                  _full((cin, ow)), _full((ow, 16)), _full((cin, 16))],
        out_specs=pl.BlockSpec((ET, W), lambda i: (i, 0)),
        out_shape=_SDS((EP, W), F32),
    )(xs, h2, w3p, smat, b3r)


def _node1_body(part, x, rootp, biasp, lin2, atts, attd, hgp, as_o, ad_o, m_o):
    agg = part[0, :, :16] + part[1, :, :16]
    h1 = jnp.maximum(agg + jnp.dot(x[...], rootp[...], preferred_element_type=F32)
                     + biasp[...], 0.0)
    hg = jnp.dot(h1, lin2[...], preferred_element_type=F32)
    hgp[...] = jnp.concatenate(
        [hg, jnp.ones((NP, 1), F32), jnp.zeros((NP, W - 17), F32)], axis=1)
    a_s = jnp.sum(hg * atts[...], axis=1, keepdims=True)
    a_d = jnp.sum(hg * attd[...], axis=1, keepdims=True)
    as_o[...] = a_s
    ad_o[...] = a_d
    m = jnp.max(a_s) + jnp.max(a_d)
    m = jnp.where(m > 0, m, 0.2 * m)
    m_o[...] = jnp.broadcast_to(m, (1, 16))


def _node1(part, xP, rootp, biasp, lin2, atts, attd):
    return pl.pallas_call(
        _node1_body,
        grid=(1,),
        in_specs=[_full((2, NP, W)), _full((NP, 128)), _full((128, 16)),
                  _full((1, 16)), _full((16, 16)), _full((1, 16)), _full((1, 16))],
        out_specs=[_full((NP, W)), _full((NP, 1)), _full((NP, 1)), _full((1, 16))],
        out_shape=[_SDS((NP, W), F32), _SDS((NP, 1), F32), _SDS((NP, 1), F32),
                   _SDS((1, 16), F32)],
    )(part, xP, rootp, biasp, lin2, atts, attd)


def _node2_body(out2, bias2, root3, bias3, h2n_o, xr3b_o):
    osum = out2[0] + out2[1]
    dinv = 1.0 / (osum[:, 16:17] + 1e-16)
    h2n = jnp.maximum(osum[:, :16] * dinv + bias2[...], 0.0)
    h2n_o[...] = jnp.concatenate([h2n, jnp.zeros((NP, W - 16), F32)], axis=1)
    xr3b_o[...] = jnp.dot(h2n, root3[...], preferred_element_type=F32) + bias3[...]


def _node2(out2, bias2, root3, bias3):
    return pl.pallas_call(
        _node2_body,
        grid=(1,),
        in_specs=[_full((2, NP, W)), _full((1, 16)),
                  _full((16, 16)), _full((1, 16))],
        out_specs=[_full((NP, W)), _full((NP, 16))],
        out_shape=[_SDS((NP, W), F32), _SDS((NP, 16), F32)],
    )(out2, bias2, root3, bias3)


def _node3_body(part, xr3b, lin4, atts, attd, hgp, as_o, ad_o, m_o):
    h3 = jnp.maximum(part[0, :, :16] + part[1, :, :16] + xr3b[...], 0.0)
    hg = jnp.dot(h3, lin4[...], preferred_element_type=F32)
    hgp[...] = jnp.concatenate(
        [hg, jnp.ones((NP, 1), F32), jnp.zeros((NP, W - 33), F32)], axis=1)
    a_s = jnp.sum(hg * atts[...], axis=1, keepdims=True)
    a_d = jnp.sum(hg * attd[...], axis=1, keepdims=True)
    as_o[...] = a_s
    ad_o[...] = a_d
    m = jnp.max(a_s) + jnp.max(a_d)
    m = jnp.where(m > 0, m, 0.2 * m)
    m_o[...] = jnp.broadcast_to(m, (1, 16))


def _node3(part, xr3b, lin4, atts, attd):
    return pl.pallas_call(
        _node3_body,
        grid=(1,),
        in_specs=[_full((2, NP, W)), _full((NP, 16)), _full((16, 32)),
                  _full((1, 32)), _full((1, 32))],
        out_specs=[_full((NP, W)), _full((NP, 1)), _full((NP, 1)), _full((1, 16))],
        out_shape=[_SDS((NP, W), F32), _SDS((NP, 1), F32), _SDS((NP, 1), F32),
                   _SDS((1, 16), F32)],
    )(part, xr3b, lin4, atts, attd)


def _head_body(out4, bias4, batch, fc1, fc1b, fc2, fc2b, res):
    osum = out4[0] + out4[1]
    dinv = 1.0 / (osum[:, 32:33] + 1e-16)
    h4 = jnp.maximum(osum[:, :32] * dinv + bias4[...], 0.0)
    gids = lax.broadcasted_iota(jnp.int32, (1, G), 1)
    oh = (batch[...] == gids).astype(F32)
    counts = jnp.sum(oh, axis=0, keepdims=True)
    pooled = lax.dot_general(oh, h4, (((0,), (0,)), ((), ())),
                             preferred_element_type=F32,
                             precision=lax.Precision.HIGHEST)
    pooled = pooled / jnp.maximum(counts, 1.0).T
    z = jnp.maximum(jnp.dot(pooled, fc1[...], preferred_element_type=F32)
                    + fc1b[...], 0.0)
    res[...] = jnp.dot(z, fc2[...], preferred_element_type=F32) + fc2b[...]


def _head(out4, bias4, batchP, fc1, fc1b, fc2, fc2b):
    return pl.pallas_call(
        _head_body,
        grid=(1,),
        in_specs=[_full((2, NP, W)), _full((1, 32)),
                  _full((NP, 1)), _full((32, 64)), _full((1, 64)),
                  _full((64, 1)), _full((1, 1))],
        out_specs=_full((G, 1)),
        out_shape=_SDS((G, 1), F32),
    )(out4, bias4, batchP, fc1, fc1b, fc2, fc2b)


@functools.lru_cache(maxsize=None)
def _mesh():
    return plsc.VectorSubcoreMesh(core_axis_name="c", subcore_axis_name="s",
                                  num_cores=2, num_subcores=16)


def _wid():
    return lax.axis_index("s") * 2 + lax.axis_index("c")


@functools.lru_cache(maxsize=None)
def _sc_gather():
    def body(table_hbm, idx_hbm, out_hbm, idxv, buf0, buf1, tsp, sem0, sem1):
        wid = _wid()
        pltpu.sync_copy(idx_hbm.at[wid], idxv)

        @pl.when(lax.axis_index("s") == 0)
        def _():
            pltpu.sync_copy(table_hbm, tsp)

        plsc.subcore_barrier()

        @pl.loop(0, NCH // 2)
        def _(ii):
            i0 = ii * 2
            i1 = i0 + 1
            cp0 = pltpu.async_copy(tsp.at[idxv.at[i0]], buf0, sem0)
            cp1 = pltpu.async_copy(tsp.at[idxv.at[i1]], buf1, sem1)
            cp0.wait()
            pltpu.sync_copy(buf0, out_hbm.at[pl.ds(wid * RPW + i0 * CH, CH)])
            cp1.wait()
            pltpu.sync_copy(buf1, out_hbm.at[pl.ds(wid * RPW + i1 * CH, CH)])

    return pl.kernel(
        body,
        out_type=_SDS((EP, W), F32),
        mesh=_mesh(),
        compiler_params=pltpu.CompilerParams(needs_layout_passes=False),
        scratch_types=[
            pltpu.VMEM((NCH, CH), jnp.int32),
            pltpu.VMEM((CH, W), F32),
            pltpu.VMEM((CH, W), F32),
            pltpu.VMEM_SHARED((NP, W), F32),
            pltpu.SemaphoreType.DMA,
            pltpu.SemaphoreType.DMA,
        ],
    )


@functools.lru_cache(maxsize=None)
def _sc_scatter_add():
    def body(vals_hbm, idx_hbm, zeros_hbm, out_hbm, idxv, buf0, buf1, accsp,
             sem0, sem1):
        wid = _wid()
        pltpu.sync_copy(idx_hbm.at[wid], idxv)

        @pl.when(lax.axis_index("s") == 0)
        def _():
            pltpu.sync_copy(zeros_hbm, accsp)

        plsc.subcore_barrier()

        @pl.loop(0, NCH // 2)
        def _(ii):
            i0 = ii * 2
            i1 = i0 + 1
            cp0 = pltpu.async_copy(vals_hbm.at[pl.ds(wid * RPW + i0 * CH, CH)],
                                   buf0, sem0)
            cp1 = pltpu.async_copy(vals_hbm.at[pl.ds(wid * RPW + i1 * CH, CH)],
                                   buf1, sem1)
            cp0.wait()
            pltpu.sync_copy(buf0, accsp.at[idxv.at[i0]], add=True)
            cp1.wait()
            pltpu.sync_copy(buf1, accsp.at[idxv.at[i1]], add=True)

        plsc.subcore_barrier()

        @pl.when(lax.axis_index("s") == 0)
        def _():
            pltpu.sync_copy(accsp, out_hbm.at[lax.axis_index("c")])

    return pl.kernel(
        body,
        out_type=_SDS((2, NP, W), F32),
        mesh=_mesh(),
        compiler_params=pltpu.CompilerParams(needs_layout_passes=False),
        scratch_types=[
            pltpu.VMEM((NCH, CH), jnp.int32),
            pltpu.VMEM((CH, W), F32),
            pltpu.VMEM((CH, W), F32),
            pltpu.VMEM_SHARED((NP, W), F32),
            pltpu.SemaphoreType.DMA,
            pltpu.SemaphoreType.DMA,
        ],
    )


@functools.lru_cache(maxsize=None)
def _sc_gat(nscale):
    def body(as_hbm, ad_hbm, hgp_hbm, m_hbm, sidx_hbm, didx_hbm, zw_hbm,
             out_hbm, asv, adv, mv, sidxv, didxv, exbuf, hrows0, osp, sem0):
        wid = _wid()
        pltpu.sync_copy(as_hbm, asv)
        pltpu.sync_copy(ad_hbm, adv)
        pltpu.sync_copy(m_hbm, mv)
        pltpu.sync_copy(sidx_hbm.at[wid], sidxv)
        pltpu.sync_copy(didx_hbm.at[wid], didxv)

        @pl.when(lax.axis_index("s") == 0)
        def _():
            pltpu.sync_copy(zw_hbm, osp)

        plsc.subcore_barrier()
        mvec = mv[0, :]

        def process(i, hrows, cp):
            srow = sidxv.at[i]
            drow = didxv.at[i]
            for j in range(CH // 16):
                sidx = srow[pl.ds(16 * j, 16)]
                didx = drow[pl.ds(16 * j, 16)]
                z = plsc.load_gather(asv, [sidx]) + plsc.load_gather(adv, [didx])
                a = jnp.where(z > 0, z, 0.2 * z)
                exbuf[pl.ds(16 * j, 16)] = jnp.exp(a - mvec)
            cp.wait()

            @pl.loop(0, CH)
            def _(r):
                cf = plsc.load_gather(exbuf, [lax.broadcast(r, (16,))])
                rrow = hrows.at[r]
                for h in range(nscale):
                    rrow[pl.ds(16 * h, 16)] = rrow[pl.ds(16 * h, 16)] * cf

            pltpu.sync_copy(hrows, osp.at[drow], add=True)

        @pl.loop(0, NCH)
        def _(i):
            cp = pltpu.async_copy(hgp_hbm.at[sidxv.at[i]], hrows0, sem0)
            process(i, hrows0, cp)

        plsc.subcore_barrier()

        @pl.when(lax.axis_index("s") == 0)
        def _():
            pltpu.sync_copy(osp, out_hbm.at[lax.axis_index("c")])

    return pl.kernel(
        body,
        out_type=_SDS((2, NP, W), F32),
        mesh=_mesh(),
        compiler_params=pltpu.CompilerParams(needs_layout_passes=False),
        scratch_types=[
            pltpu.VMEM((NP,), F32),
            pltpu.VMEM((NP,), F32),
            pltpu.VMEM((1, 16), F32),
            pltpu.VMEM((NCH, CH), jnp.int32),
            pltpu.VMEM((NCH, CH), jnp.int32),
            pltpu.VMEM((CH,), F32),
            pltpu.VMEM((CH, W), F32),
            pltpu.VMEM_SHARED((NP, W), F32),
            pltpu.SemaphoreType.DMA,
        ],
    )


def _gather_rows(*a):
    return _sc_gather()(*a)


def _scatter_rows(*a):
    return _sc_scatter_add()(*a)


def _gat2pass(*a):
    return _sc_gat(2)(*a)


def _gat3pass(*a):
    return _sc_gat(3)(*a)


def _w3_omajor(w3, cin, cout):
    return jnp.reshape(jnp.transpose(jnp.reshape(w3, (64, cin, cout)),
                                     (1, 2, 0)), (cin, cout * 64))


def _smat(cout):
    s = np.kron(np.eye(cout, dtype=np.float32), np.ones((64, 1), np.float32))
    return jnp.asarray(np.pad(s, ((0, 0), (0, 16 - cout))))


def _b3r(b3, cin, cout):
    return jnp.pad(jnp.reshape(b3, (cin, cout)), ((0, 0), (0, 16 - cout)))


def kernel(x, edge_index, edge_attr, batch, emlp1_w1, emlp1_b1, emlp1_w2,
           emlp1_b2, emlp1_w3, emlp1_b3, conv1_root, conv1_bias, gat2_lin,
           gat2_att_src, gat2_att_dst, gat2_bias, emlp2_w1, emlp2_b1,
           emlp2_w2, emlp2_b2, emlp2_w3, emlp2_b3, conv3_root, conv3_bias,
           gat4_lin, gat4_att_src, gat4_att_dst, gat4_bias, fc1_w, fc1_b,
           fc2_w, fc2_b):
    xP = jnp.pad(x, ((0, NP - N), (0, 0)))
    eaP = jnp.pad(edge_attr, ((0, EP - E), (0, 0)))
    src = jnp.pad(edge_index[0], (0, EP - E), constant_values=N)
    dst = jnp.pad(edge_index[1], (0, EP - E), constant_values=N)
    sidx3 = jnp.reshape(src, (NW, NCH, CH))
    didx3 = jnp.reshape(dst, (NW, NCH, CH))
    batchP = jnp.pad(batch, (0, NP - N), constant_values=G)[:, None]

    zw = jnp.zeros((NP, W), F32)

    w3p1 = _w3_omajor(emlp1_w3, 128, 8)
    s1 = _smat(8)
    b3r1 = _b3r(emlp1_b3, 128, 8)
    w3p2 = _w3_omajor(emlp2_w3, 16, 16)
    s2 = _smat(16)
    b3r2 = _b3r(emlp2_b3, 16, 16)
    root1p = jnp.pad(conv1_root, ((0, 0), (0, 8)))
    bias1p = jnp.pad(conv1_bias, (0, 8))[None, :]
    lin2p = jnp.pad(gat2_lin, ((0, 8), (0, 0)))

    h2a, h2b = _edge_mlp(eaP, emlp1_w1, emlp1_b1[None, :], emlp1_w2,
                         emlp1_b2[None, :], emlp2_w1, emlp2_b1[None, :],
                         emlp2_w2, emlp2_b2[None, :])
    xs1 = _gather_rows(xP, sidx3)

    msg1 = _conv_edge(xs1, h2a, w3p1, s1, b3r1, 128, 8)
    part1 = _scatter_rows(msg1, didx3, zw)
    hgp2, as2, ad2, m2 = _node1(part1, xP, root1p, bias1p, lin2p,
                                gat2_att_src[None, :], gat2_att_dst[None, :])

    out2 = _gat2pass(jnp.reshape(as2, (NP,)), jnp.reshape(ad2, (NP,)),
                     hgp2, m2, sidx3, didx3, zw)
    h2n, xr3b = _node2(out2, gat2_bias[None, :], conv3_root,
                       conv3_bias[None, :])

    xs3 = _gather_rows(h2n, sidx3)
    msg3 = _conv_edge(xs3, h2b, w3p2, s2, b3r2, 16, 16)
    part3 = _scatter_rows(msg3, didx3, zw)
    hgp4, as4, ad4, m4 = _node3(part3, xr3b, gat4_lin,
                                gat4_att_src[None, :], gat4_att_dst[None, :])

    out4 = _gat3pass(jnp.reshape(as4, (NP,)), jnp.reshape(ad4, (NP,)),
                     hgp4, m4, sidx3, didx3, zw)
    res = _head(out4, gat4_bias[None, :], batchP, fc1_w, fc1_b[None, :],
                fc2_w, jnp.reshape(fc2_b, (1, 1)))
    return jnp.reshape(res, (G,))
